# baseline (speedup 1.0000x reference)
import functools

import jax
import jax.numpy as jnp
from jax.experimental import pallas as pl
from jax.experimental.pallas import tpu as pltpu
from jax.experimental.pallas import tpu_sc as plsc

_N = 8192
_K = 8192
_D = 32
_BETA = 0.5

_TM = 256
_TK = 2048


def _dist_argmin_body(a_ref, b_ref, z_ref, w_ref, idx_ref, loss_ref,
                      rmin_ref, ridx_ref):
    k = pl.program_id(1)
    nk = pl.num_programs(1)

    c = jax.lax.dot_general(
        z_ref[...], w_ref[...], (((1,), (1,)), ((), ())),
        preferred_element_type=jnp.float32)
    d = (a_ref[...] + b_ref[...]) - 2.0 * c

    bmin = jnp.min(d, axis=1, keepdims=True)
    col = jax.lax.broadcasted_iota(jnp.int32, d.shape, 1) + k * _TK
    bidx = jnp.min(jnp.where(d == bmin, col, jnp.int32(2 ** 30)),
                   axis=1, keepdims=True)

    @pl.when(jnp.logical_and(k == 0, pl.program_id(0) == 0))
    def _():
        loss_ref[...] = jnp.zeros((1, 1), jnp.float32)

    @pl.when(k == 0)
    def _():
        rmin_ref[...] = bmin.astype(jnp.bfloat16).astype(jnp.float32)
        ridx_ref[...] = bidx

    @pl.when(k > 0)
    def _():
        better = bmin < rmin_ref[...]
        merged = jnp.where(better, bmin, rmin_ref[...])
        rmin_ref[...] = merged.astype(jnp.bfloat16).astype(jnp.float32)
        ridx_ref[...] = jnp.where(better, bidx, ridx_ref[...])

    @pl.when(k == nk - 1)
    def _():
        idx_ref[...] = ridx_ref[...]
        loss_ref[...] += jnp.sum(rmin_ref[...]).reshape(1, 1)


def _dist_argmin(z, w, a, b):
    grid = (_N // _TM, _K // _TK)
    return pl.pallas_call(
        _dist_argmin_body,
        grid=grid,
        in_specs=[
            pl.BlockSpec((_TM, 1), lambda i, k: (i, 0)),
            pl.BlockSpec((1, _TK), lambda i, k: (0, k)),
            pl.BlockSpec((_TM, _D), lambda i, k: (i, 0)),
            pl.BlockSpec((_TK, _D), lambda i, k: (k, 0)),
        ],
        out_specs=[
            pl.BlockSpec((_TM, 1), lambda i, k: (i, 0)),
            pl.BlockSpec((1, 1), lambda i, k: (0, 0)),
        ],
        out_shape=[
            jax.ShapeDtypeStruct((_N, 1), jnp.int32),
            jax.ShapeDtypeStruct((1, 1), jnp.float32),
        ],
        scratch_shapes=[
            pltpu.VMEM((_TM, 1), jnp.float32),
            pltpu.VMEM((_TM, 1), jnp.int32),
        ],
    )(a, b, z, w)


_NC = 2
_NS = 16
_NW = _NC * _NS
_BPW = _N // _NW


_DP = 128
_NCH = _BPW // 128


@functools.cache
def _sc_gather_kernel():
    mesh = plsc.VectorSubcoreMesh(core_axis_name="c", subcore_axis_name="s")

    @functools.partial(
        pl.kernel,
        mesh=mesh,
        out_type=jax.ShapeDtypeStruct((_N, _DP), jnp.float32),
        scratch_types=[
            pltpu.VMEM((_NCH, 128), jnp.int32),
            pltpu.VMEM((_BPW, _DP), jnp.float32),
            pltpu.SemaphoreType.DMA,
        ],
    )
    def sc_gather(table_hbm, idx_hbm, out_hbm, idx_v, rows_v, sem):
        wid = jax.lax.axis_index("s") * _NC + jax.lax.axis_index("c")
        base = wid * _BPW
        pltpu.sync_copy(idx_hbm.at[wid], idx_v)
        for j in range(_NCH):
            pltpu.async_copy(table_hbm.at[idx_v.at[j]],
                             rows_v.at[pl.ds(j * 128, 128)], sem)
        for j in range(_NCH):
            pltpu.make_async_copy(table_hbm.at[idx_v.at[j]],
                                  rows_v.at[pl.ds(j * 128, 128)], sem).wait()
        pltpu.sync_copy(rows_v, out_hbm.at[pl.ds(base, _BPW)])

    return sc_gather


def kernel(z, W):
    a = jnp.sum(z ** 2, axis=1, keepdims=True)
    b = jnp.sum(W ** 2, axis=1).reshape(1, _K)
    idx2d, loss_sum = _dist_argmin(z, W, a, b)
    indices = idx2d.reshape(_N)
    w_pad = jnp.pad(W, ((0, 0), (0, _DP - _D)))
    zq_pad = _sc_gather_kernel()(w_pad, idx2d.reshape(_NW, _NCH, 128))
    zq = zq_pad[:, :_D]
    m = loss_sum.reshape(()) / (_N * _D)
    loss = _BETA * m + m
    zq_out = z + jax.lax.stop_gradient(zq - z)
    return (zq_out, indices, loss)


# speedup vs baseline: 1.5077x; 1.5077x over previous
import functools

import jax
import jax.numpy as jnp
from jax.experimental import pallas as pl
from jax.experimental.pallas import tpu as pltpu
from jax.experimental.pallas import tpu_sc as plsc

_N = 8192
_K = 8192
_D = 32
_BETA = 0.5

_TM = 512
_TK = 2048


def _dist_argmin_body(a_ref, b_ref, z_ref, w_ref, idx_ref, loss_ref,
                      rmin_ref, ridx_ref, c_ref):
    k = pl.program_id(1)
    nk = pl.num_programs(1)

    c_ref[...] = jax.lax.dot_general(
        w_ref[...], z_ref[...], (((1,), (1,)), ((), ())),
        preferred_element_type=jnp.float32)
    a = a_ref[...]
    kbase = k * _TK

    def scan_body(j, carry):
        vval, vidx = carry
        cs = c_ref[pl.ds(8 * j, 8), :]
        bs = b_ref[pl.ds(8 * j, 8), :]
        d = (a + bs) - 2.0 * cs
        kk = (kbase + 8 * j
              + jax.lax.broadcasted_iota(jnp.int32, (8, _TM), 0))
        lt = d < vval
        return jnp.where(lt, d, vval), jnp.where(lt, kk, vidx)

    init = (jnp.full((8, _TM), jnp.inf, jnp.float32),
            jnp.zeros((8, _TM), jnp.int32))
    vval, vidx = jax.lax.fori_loop(0, _TK // 8, scan_body, init,
                                   unroll=True)

    bmin = jnp.min(vval, axis=0, keepdims=True)
    bidx = jnp.min(jnp.where(vval == bmin, vidx, jnp.int32(2 ** 30)),
                   axis=0, keepdims=True)

    @pl.when(jnp.logical_and(k == 0, pl.program_id(0) == 0))
    def _():
        loss_ref[...] = jnp.zeros((1, 1), jnp.float32)

    @pl.when(k == 0)
    def _():
        rmin_ref[...] = bmin.astype(jnp.bfloat16).astype(jnp.float32)
        ridx_ref[...] = bidx

    @pl.when(k > 0)
    def _():
        better = bmin < rmin_ref[...]
        merged = jnp.where(better, bmin, rmin_ref[...])
        rmin_ref[...] = merged.astype(jnp.bfloat16).astype(jnp.float32)
        ridx_ref[...] = jnp.where(better, bidx, ridx_ref[...])

    @pl.when(k == nk - 1)
    def _():
        idx_ref[...] = ridx_ref[...].reshape(1, 1, _TM)
        loss_ref[...] += jnp.sum(rmin_ref[...]).reshape(1, 1)


def _dist_argmin(z, w, a, b):
    grid = (_N // _TM, _K // _TK)
    return pl.pallas_call(
        _dist_argmin_body,
        grid=grid,
        in_specs=[
            pl.BlockSpec((1, _TM), lambda i, k: (0, i)),
            pl.BlockSpec((_TK, 1), lambda i, k: (k, 0)),
            pl.BlockSpec((_TM, _D), lambda i, k: (i, 0)),
            pl.BlockSpec((_TK, _D), lambda i, k: (k, 0)),
        ],
        out_specs=[
            pl.BlockSpec((1, 1, _TM), lambda i, k: (i, 0, 0)),
            pl.BlockSpec((1, 1), lambda i, k: (0, 0)),
        ],
        out_shape=[
            jax.ShapeDtypeStruct((_N // _TM, 1, _TM), jnp.int32),
            jax.ShapeDtypeStruct((1, 1), jnp.float32),
        ],
        scratch_shapes=[
            pltpu.VMEM((1, _TM), jnp.float32),
            pltpu.VMEM((1, _TM), jnp.int32),
            pltpu.VMEM((_TK, _TM), jnp.float32),
        ],
    )(a, b, z, w)


_NC = 2
_NS = 16
_NW = _NC * _NS
_BPW = _N // _NW


_DP = 128
_NCH = _BPW // 128


@functools.cache
def _sc_gather_kernel():
    mesh = plsc.VectorSubcoreMesh(core_axis_name="c", subcore_axis_name="s")

    @functools.partial(
        pl.kernel,
        mesh=mesh,
        out_type=jax.ShapeDtypeStruct((_N, _DP), jnp.float32),
        scratch_types=[
            pltpu.VMEM((_NCH, 128), jnp.int32),
            pltpu.VMEM((_BPW, _DP), jnp.float32),
            pltpu.SemaphoreType.DMA,
        ],
    )
    def sc_gather(table_hbm, idx_hbm, out_hbm, idx_v, rows_v, sem):
        wid = jax.lax.axis_index("s") * _NC + jax.lax.axis_index("c")
        base = wid * _BPW
        pltpu.sync_copy(idx_hbm.at[wid], idx_v)
        for j in range(_NCH):
            pltpu.async_copy(table_hbm.at[idx_v.at[j]],
                             rows_v.at[pl.ds(j * 128, 128)], sem)
        for j in range(_NCH):
            pltpu.make_async_copy(table_hbm.at[idx_v.at[j]],
                                  rows_v.at[pl.ds(j * 128, 128)], sem).wait()
        pltpu.sync_copy(rows_v, out_hbm.at[pl.ds(base, _BPW)])

    return sc_gather


def kernel(z, W):
    a = jnp.sum(z ** 2, axis=1).reshape(1, _N)
    b = jnp.sum(W ** 2, axis=1).reshape(_K, 1)
    idx3d, loss_sum = _dist_argmin(z, W, a, b)
    idx2d = idx3d.reshape(_N, 1)
    indices = idx2d.reshape(_N)
    w_pad = jnp.pad(W, ((0, 0), (0, _DP - _D)))
    zq_pad = _sc_gather_kernel()(w_pad, idx2d.reshape(_NW, _NCH, 128))
    zq = zq_pad[:, :_D]
    m = loss_sum.reshape(()) / (_N * _D)
    loss = _BETA * m + m
    zq_out = z + jax.lax.stop_gradient(zq - z)
    return (zq_out, indices, loss)


# speedup vs baseline: 2.0854x; 1.3832x over previous
import functools

import jax
import jax.numpy as jnp
from jax.experimental import pallas as pl
from jax.experimental.pallas import tpu as pltpu
from jax.experimental.pallas import tpu_sc as plsc

_N = 8192
_K = 8192
_D = 32
_BETA = 0.5

_TM = 1024
_TK = 2048
_NCHUNK = _K // _TK


def _dist_argmin_body(a_ref, b_ref, z_ref, w2_ref, idx_ref, loss_ref,
                      c0_ref, c1_ref):
    i = pl.program_id(0)
    a = a_ref[...]
    z = z_ref[...]

    rmin = None
    ridx = None
    for ch in range(_NCHUNK):
        c_ref = (c0_ref, c1_ref)[ch % 2]
        c_ref[...] = jax.lax.dot_general(
            w2_ref[pl.ds(ch * _TK, _TK), :], z, (((1,), (1,)), ((), ())),
            preferred_element_type=jnp.float32)
        kbase = ch * _TK

        def scan_body(j, carry, c_ref=c_ref, kbase=kbase):
            vval, vidx = carry
            cs2 = c_ref[pl.ds(8 * j, 8), :]
            bs = b_ref[pl.ds(kbase + 8 * j, 8), :]
            d = (a + bs) - cs2
            kk = (kbase + 8 * j
                  + jax.lax.broadcasted_iota(jnp.int32, (8, _TM), 0))
            lt = d < vval
            return jnp.where(lt, d, vval), jnp.where(lt, kk, vidx)

        init = (jnp.full((8, _TM), jnp.inf, jnp.float32),
                jnp.zeros((8, _TM), jnp.int32))
        vval, vidx = jax.lax.fori_loop(0, _TK // 8, scan_body, init,
                                       unroll=True)

        bmin = jnp.min(vval, axis=0, keepdims=True)
        bidx = jnp.min(jnp.where(vval == bmin, vidx, jnp.int32(2 ** 30)),
                       axis=0, keepdims=True)

        if ch == 0:
            rmin = bmin.astype(jnp.bfloat16).astype(jnp.float32)
            ridx = bidx
        else:
            better = bmin < rmin
            rmin = jnp.where(better, bmin, rmin)
            rmin = rmin.astype(jnp.bfloat16).astype(jnp.float32)
            ridx = jnp.where(better, bidx, ridx)

    idx_ref[...] = ridx.reshape(1, 1, _TM)
    lsum = jnp.sum(rmin).reshape(1, 1)

    @pl.when(i == 0)
    def _():
        loss_ref[...] = lsum

    @pl.when(i > 0)
    def _():
        loss_ref[...] += lsum


def _dist_argmin(z, w2, a, b):
    grid = (_N // _TM,)
    return pl.pallas_call(
        _dist_argmin_body,
        grid=grid,
        in_specs=[
            pl.BlockSpec((1, _TM), lambda i: (0, i)),
            pl.BlockSpec((_K, 1), lambda i: (0, 0)),
            pl.BlockSpec((_TM, _D), lambda i: (i, 0)),
            pl.BlockSpec((_K, _D), lambda i: (0, 0)),
        ],
        out_specs=[
            pl.BlockSpec((1, 1, _TM), lambda i: (i, 0, 0)),
            pl.BlockSpec((1, 1), lambda i: (0, 0)),
        ],
        out_shape=[
            jax.ShapeDtypeStruct((_N // _TM, 1, _TM), jnp.int32),
            jax.ShapeDtypeStruct((1, 1), jnp.float32),
        ],
        scratch_shapes=[
            pltpu.VMEM((_TK, _TM), jnp.float32),
            pltpu.VMEM((_TK, _TM), jnp.float32),
        ],
    )(a, b, z, w2)


_NC = 2
_NS = 16
_NW = _NC * _NS
_BPW = _N // _NW


_DP = 128
_NCH = _BPW // 128


@functools.cache
def _sc_gather_kernel():
    mesh = plsc.VectorSubcoreMesh(core_axis_name="c", subcore_axis_name="s")

    @functools.partial(
        pl.kernel,
        mesh=mesh,
        out_type=jax.ShapeDtypeStruct((_N, _DP), jnp.float32),
        scratch_types=[
            pltpu.VMEM((_NCH, 128), jnp.int32),
            pltpu.VMEM((_BPW, _DP), jnp.float32),
            pltpu.SemaphoreType.DMA,
        ],
    )
    def sc_gather(table_hbm, idx_hbm, out_hbm, idx_v, rows_v, sem):
        wid = jax.lax.axis_index("s") * _NC + jax.lax.axis_index("c")
        base = wid * _BPW
        pltpu.sync_copy(idx_hbm.at[wid], idx_v)
        for j in range(_NCH):
            pltpu.async_copy(table_hbm.at[idx_v.at[j]],
                             rows_v.at[pl.ds(j * 128, 128)], sem)
        for j in range(_NCH):
            pltpu.make_async_copy(table_hbm.at[idx_v.at[j]],
                                  rows_v.at[pl.ds(j * 128, 128)], sem).wait()
        pltpu.sync_copy(rows_v, out_hbm.at[pl.ds(base, _BPW)])

    return sc_gather


def kernel(z, W):
    a = jnp.sum(z ** 2, axis=1).reshape(1, _N)
    b = jnp.sum(W ** 2, axis=1).reshape(_K, 1)
    w2 = W + W
    idx3d, loss_sum = _dist_argmin(z, w2, a, b)
    indices = idx3d.reshape(_N)
    w_pad = jnp.pad(W, ((0, 0), (0, _DP - _D)))
    zq_pad = _sc_gather_kernel()(w_pad, indices.reshape(_NW, _NCH, 128))
    zq = zq_pad[:, :_D]
    m = loss_sum.reshape(()) / (_N * _D)
    loss = _BETA * m + m
    return (zq, indices, loss)


# speedup vs baseline: 2.1770x; 1.0439x over previous
import functools

import jax
import jax.numpy as jnp
from jax.experimental import pallas as pl
from jax.experimental.pallas import tpu as pltpu
from jax.experimental.pallas import tpu_sc as plsc

_N = 8192
_K = 8192
_D = 32
_BETA = 0.5

_TM = 2048
_TK = 2048
_NCHUNK = _K // _TK


def _dist_argmin_body(a_ref, b_ref, z_ref, w_ref, idx_ref, loss_ref,
                      c0_ref, c1_ref, w2_ref):
    i = pl.program_id(0)
    a = a_ref[...]
    z = z_ref[...]

    @pl.when(i == 0)
    def _():
        w2_ref[...] = w_ref[...] + w_ref[...]

    rmin = None
    ridx = None
    for ch in range(_NCHUNK):
        c_ref = (c0_ref, c1_ref)[ch % 2]
        c_ref[...] = jax.lax.dot_general(
            w2_ref[pl.ds(ch * _TK, _TK), :], z, (((1,), (1,)), ((), ())),
            preferred_element_type=jnp.float32)
        kbase = ch * _TK

        def scan_body(j, carry, c_ref=c_ref, kbase=kbase):
            vval, vidx = carry
            cs2 = c_ref[pl.ds(8 * j, 8), :]
            bs = b_ref[pl.ds(kbase + 8 * j, 8), :]
            d = (a + bs) - cs2
            kk = (kbase + 8 * j
                  + jax.lax.broadcasted_iota(jnp.int32, (8, _TM), 0))
            lt = d < vval
            return jnp.where(lt, d, vval), jnp.where(lt, kk, vidx)

        init = (jnp.full((8, _TM), jnp.inf, jnp.float32),
                jnp.zeros((8, _TM), jnp.int32))
        vval, vidx = jax.lax.fori_loop(0, _TK // 8, scan_body, init,
                                       unroll=True)

        bmin = jnp.min(vval, axis=0, keepdims=True)
        bidx = jnp.min(jnp.where(vval == bmin, vidx, jnp.int32(2 ** 30)),
                       axis=0, keepdims=True)

        if ch == 0:
            rmin = bmin.astype(jnp.bfloat16).astype(jnp.float32)
            ridx = bidx
        else:
            better = bmin < rmin
            rmin = jnp.where(better, bmin, rmin)
            rmin = rmin.astype(jnp.bfloat16).astype(jnp.float32)
            ridx = jnp.where(better, bidx, ridx)

    idx_ref[...] = ridx.reshape(1, 1, _TM)
    lsum = jnp.sum(rmin).reshape(1, 1)

    @pl.when(i == 0)
    def _():
        loss_ref[...] = lsum

    @pl.when(i > 0)
    def _():
        loss_ref[...] += lsum


def _dist_argmin(z, w, a, b):
    grid = (_N // _TM,)
    return pl.pallas_call(
        _dist_argmin_body,
        grid=grid,
        in_specs=[
            pl.BlockSpec((1, _TM), lambda i: (0, i)),
            pl.BlockSpec((_K, 1), lambda i: (0, 0)),
            pl.BlockSpec((_TM, _D), lambda i: (i, 0)),
            pl.BlockSpec((_K, _D), lambda i: (0, 0)),
        ],
        out_specs=[
            pl.BlockSpec((1, 1, _TM), lambda i: (i, 0, 0)),
            pl.BlockSpec((1, 1), lambda i: (0, 0)),
        ],
        out_shape=[
            jax.ShapeDtypeStruct((_N // _TM, 1, _TM), jnp.int32),
            jax.ShapeDtypeStruct((1, 1), jnp.float32),
        ],
        scratch_shapes=[
            pltpu.VMEM((_TK, _TM), jnp.float32),
            pltpu.VMEM((_TK, _TM), jnp.float32),
            pltpu.VMEM((_K, _D), jnp.float32),
        ],
    )(a, b, z, w)


_NC = 2
_NS = 16
_NW = _NC * _NS
_BPW = _N // _NW


_DP = 128
_NCH = _BPW // 128


@functools.cache
def _sc_gather_kernel():
    mesh = plsc.VectorSubcoreMesh(core_axis_name="c", subcore_axis_name="s")

    @functools.partial(
        pl.kernel,
        mesh=mesh,
        out_type=jax.ShapeDtypeStruct((_N, _DP), jnp.float32),
        scratch_types=[
            pltpu.VMEM((_NCH, 128), jnp.int32),
            pltpu.VMEM((_BPW, _DP), jnp.float32),
            pltpu.SemaphoreType.DMA,
        ],
    )
    def sc_gather(table_hbm, idx_hbm, out_hbm, idx_v, rows_v, sem):
        wid = jax.lax.axis_index("s") * _NC + jax.lax.axis_index("c")
        base = wid * _BPW
        pltpu.sync_copy(idx_hbm.at[wid], idx_v)
        for j in range(_NCH):
            pltpu.async_copy(table_hbm.at[idx_v.at[j]],
                             rows_v.at[pl.ds(j * 128, 128)], sem)
        for j in range(_NCH):
            pltpu.make_async_copy(table_hbm.at[idx_v.at[j]],
                                  rows_v.at[pl.ds(j * 128, 128)], sem).wait()
        pltpu.sync_copy(rows_v, out_hbm.at[pl.ds(base, _BPW)])

    return sc_gather


def kernel(z, W):
    a = jnp.sum(z ** 2, axis=1).reshape(1, _N)
    b = jnp.sum(W ** 2, axis=1).reshape(_K, 1)
    idx3d, loss_sum = _dist_argmin(z, W, a, b)
    indices = idx3d.reshape(_N)
    w_pad = jnp.pad(W, ((0, 0), (0, _DP - _D)))
    zq_pad = _sc_gather_kernel()(w_pad, indices.reshape(_NW, _NCH, 128))
    zq = zq_pad[:, :_D]
    m = loss_sum.reshape(()) / (_N * _D)
    loss = _BETA * m + m
    return (zq, indices, loss)
